# baseline (speedup 1.0000x reference)
import functools

import jax
import jax.numpy as jnp
from jax import lax
from jax.experimental import pallas as pl
from jax.experimental.pallas import tpu as pltpu
from jax.experimental.pallas import tpu_sc as plsc

EMBED_DIM = 768
FFN_DIM = 1536
NUM_EXPERTS = 8
CAPACITY = 512
ALPHA = 0.01
N_TOK = 2048
LANES = 128
TOK_TILE = 512
N_E = NUM_EXPERTS + 1
GRID = NUM_EXPERTS + N_TOK // TOK_TILE
OW_COL = 2 * NUM_EXPERTS


def _router_body(x_ref, rw_ref, rb_ref, wv_ref, aux_ref):
    x = x_ref[...]
    logits = jnp.dot(x, rw_ref[...], preferred_element_type=jnp.float32)
    logits = logits + rb_ref[...]
    col = jax.lax.broadcasted_iota(jnp.int32, (N_TOK, LANES), 1)
    valid = col < NUM_EXPERTS
    logits = jnp.where(valid, logits, jnp.float32(-1e30))
    m = jnp.max(logits, axis=1, keepdims=True)
    ex = jnp.where(valid, jnp.exp(logits - m), 0.0)
    s = ex / jnp.sum(ex, axis=1, keepdims=True)
    s = jnp.where(valid, s, -1.0)
    big_i = jnp.int32(LANES)
    m1 = jnp.max(s, axis=1, keepdims=True)
    i1 = jnp.min(jnp.where(s == m1, col, big_i), axis=1, keepdims=True)
    s2m = jnp.where(col == i1, -2.0, s)
    m2 = jnp.max(s2m, axis=1, keepdims=True)
    i2 = jnp.min(jnp.where(s2m == m2, col, big_i), axis=1, keepdims=True)
    wsum = m1 + m2
    wfull = (jnp.where(col == i1, m1 / wsum, 0.0)
             + jnp.where(col == i2, m2 / wsum, 0.0))
    maskf = jnp.where((col == i1) | (col == i2), 1.0, 0.0)
    rank = maskf
    k = 1
    while k < N_TOK:
        shifted = jnp.concatenate(
            [jnp.zeros((k, LANES), jnp.float32), rank[: N_TOK - k, :]], axis=0)
        rank = rank + shifted
        k *= 2
    within = (maskf > 0.0) & (rank <= jnp.float32(CAPACITY))
    overflow = (maskf > 0.0) & ~within
    w_within = jnp.where(within, wfull, 0.0)
    ow_tok = jnp.sum(jnp.where(overflow, wfull, 0.0), axis=1, keepdims=True)
    nw = jnp.sum(within.astype(jnp.float32), axis=0, keepdims=True)
    p = jnp.sum(w_within, axis=0, keepdims=True) / jnp.maximum(nw, 1.0)
    f = nw / jnp.float32(N_TOK)
    aux_ref[...] = jnp.float32(ALPHA * NUM_EXPERTS) * jnp.sum(
        f * p, keepdims=True)
    col8 = col - NUM_EXPERTS
    destf = jnp.where(within, rank - 1.0, -1.0)
    dest_sh = jnp.where((col8 >= 0) & (col8 < NUM_EXPERTS),
                        jnp.roll(destf, NUM_EXPERTS, axis=1), 0.0)
    wv_ref[...] = (w_within + dest_sh
                   + jnp.where(col == OW_COL, ow_tok, 0.0))


def _sc_dispatch(wvT_hbm, idx_hbm, wbuf_hbm,
                 wvec, dvec, idxv, wslotv):
    core = lax.axis_index("c")
    sub = lax.axis_index("s")
    lane = lax.iota(jnp.int32, 16)

    @pl.when(sub < 4)
    def _pack():
        e = core * 4 + sub
        pltpu.sync_copy(wvT_hbm.at[e], wvec)
        pltpu.sync_copy(wvT_hbm.at[NUM_EXPERTS + e], dvec)
        zi = jnp.zeros((16,), jnp.int32)
        zf = jnp.zeros((16,), jnp.float32)
        for i in range(CAPACITY // 16):
            idxv[pl.ds(i * 16, 16)] = zi
            wslotv[pl.ds(i * 16, 16)] = zf

        def body(i, carry):
            d16 = dvec[pl.ds(i * 16, 16)]
            within = d16 >= 0.0
            dest = jnp.maximum(d16, 0.0).astype(jnp.int32)
            w16 = wvec[pl.ds(i * 16, 16)]
            tok = lane + i * 16
            plsc.store_scatter(idxv, [dest], tok, mask=within)
            plsc.store_scatter(wslotv, [dest], w16, mask=within)
            return carry

        lax.fori_loop(0, N_TOK // 16, body, 0)
        pltpu.sync_copy(idxv, idx_hbm.at[e])
        pltpu.sync_copy(wslotv, wbuf_hbm.at[e])


def _make_dispatch():
    return pl.kernel(
        _sc_dispatch,
        out_type=[
            jax.ShapeDtypeStruct((NUM_EXPERTS, CAPACITY), jnp.int32),
            jax.ShapeDtypeStruct((NUM_EXPERTS, CAPACITY), jnp.float32),
        ],
        mesh=plsc.VectorSubcoreMesh(core_axis_name="c", subcore_axis_name="s"),
        compiler_params=pltpu.CompilerParams(needs_layout_passes=False),
        scratch_types=[
            pltpu.VMEM((N_TOK,), jnp.float32),
            pltpu.VMEM((N_TOK,), jnp.float32),
            pltpu.VMEM((CAPACITY,), jnp.int32),
            pltpu.VMEM((CAPACITY,), jnp.float32),
        ],
    )


def _ovf_body(xf_ref, ow1_ref, ob1_ref, ow2_ref, ob2_ref, owT_ref, out_ref):
    x = xf_ref[...].astype(jnp.bfloat16)
    h = jnp.dot(x, ow1_ref[...].astype(jnp.bfloat16),
                preferred_element_type=jnp.float32)
    h = jax.nn.relu(h + ob1_ref[...])
    y = jnp.dot(h.astype(jnp.bfloat16), ow2_ref[...].astype(jnp.bfloat16),
                preferred_element_type=jnp.float32)
    y = y + ob2_ref[...]
    c = jnp.sum(owT_ref[...], axis=1, keepdims=True)
    out_ref[...] = c * y


def _exp_body(xf_ref, w1_ref, b1_ref, w2_ref, b2_ref, idx_ref,
              wbT_ref, ovf_ref, out_ref):
    i = pl.program_id(0)

    @pl.when(i == 0)
    def _():
        out_ref[...] = ovf_ref[...]

    trow = jax.lax.broadcasted_iota(jnp.int32, (N_TOK, CAPACITY), 0)
    oh = (trow == idx_ref[0]).astype(jnp.bfloat16)
    x = jax.lax.dot_general(
        oh, xf_ref[...].astype(jnp.bfloat16),
        dimension_numbers=(((0,), (0,)), ((), ())),
        preferred_element_type=jnp.float32).astype(jnp.bfloat16)
    h = jnp.dot(x, w1_ref[0].astype(jnp.bfloat16),
                preferred_element_type=jnp.float32)
    h = jax.nn.relu(h + b1_ref[0])
    y = jnp.dot(h.astype(jnp.bfloat16), w2_ref[0].astype(jnp.bfloat16),
                preferred_element_type=jnp.float32)
    y = y + b2_ref[0]
    lane8 = jax.lax.broadcasted_iota(jnp.int32, (CAPACITY, NUM_EXPERTS), 1)
    wcol = jnp.sum(jnp.where(lane8 == i, wbT_ref[...], 0.0),
                   axis=1, keepdims=True)
    ys = (wcol * y).astype(jnp.bfloat16)
    out_ref[...] += jnp.dot(oh, ys, preferred_element_type=jnp.float32)


def kernel(input, router_w, router_b, w1, b1, w2, b2, ow1, ob1, ow2, ob2):
    flat = input.reshape(N_TOK, EMBED_DIM)
    rw_p = jnp.pad(router_w, ((0, 0), (0, LANES - NUM_EXPERTS)))
    rb_p = jnp.pad(router_b.reshape(1, NUM_EXPERTS),
                   ((0, 0), (0, LANES - NUM_EXPERTS)))

    wv, aux = pl.pallas_call(
        _router_body,
        out_shape=[
            jax.ShapeDtypeStruct((N_TOK, LANES), jnp.float32),
            jax.ShapeDtypeStruct((1, 1), jnp.float32),
        ],
    )(flat, rw_p, rb_p)

    wvT = wv[:, :2 * NUM_EXPERTS].T

    idx, wbuf = _make_dispatch()(wvT)

    idx3 = idx.reshape(NUM_EXPERTS, 1, CAPACITY)
    wbT = wbuf.T
    owT = wv[:, OW_COL:OW_COL + NUM_EXPERTS]

    b1r = b1.reshape(NUM_EXPERTS, 1, FFN_DIM)
    b2r = b2.reshape(NUM_EXPERTS, 1, EMBED_DIM)
    ob1r = ob1.reshape(1, FFN_DIM)
    ob2r = ob2.reshape(1, EMBED_DIM)

    ovf = pl.pallas_call(
        _ovf_body,
        grid=(N_TOK // TOK_TILE,),
        in_specs=[
            pl.BlockSpec((TOK_TILE, EMBED_DIM), lambda t: (t, 0)),
            pl.BlockSpec((EMBED_DIM, FFN_DIM), lambda t: (0, 0)),
            pl.BlockSpec((1, FFN_DIM), lambda t: (0, 0)),
            pl.BlockSpec((FFN_DIM, EMBED_DIM), lambda t: (0, 0)),
            pl.BlockSpec((1, EMBED_DIM), lambda t: (0, 0)),
            pl.BlockSpec((TOK_TILE, NUM_EXPERTS), lambda t: (t, 0)),
        ],
        out_specs=pl.BlockSpec((TOK_TILE, EMBED_DIM), lambda t: (t, 0)),
        out_shape=jax.ShapeDtypeStruct((N_TOK, EMBED_DIM), jnp.float32),
    )(flat, ow1, ob1r, ow2, ob2r, owT)

    out = pl.pallas_call(
        _exp_body,
        grid=(NUM_EXPERTS,),
        in_specs=[
            pl.BlockSpec((N_TOK, EMBED_DIM), lambda i: (0, 0)),
            pl.BlockSpec((1, EMBED_DIM, FFN_DIM), lambda i: (i, 0, 0)),
            pl.BlockSpec((1, 1, FFN_DIM), lambda i: (i, 0, 0)),
            pl.BlockSpec((1, FFN_DIM, EMBED_DIM), lambda i: (i, 0, 0)),
            pl.BlockSpec((1, 1, EMBED_DIM), lambda i: (i, 0, 0)),
            pl.BlockSpec((1, 1, CAPACITY), lambda i: (i, 0, 0)),
            pl.BlockSpec((CAPACITY, NUM_EXPERTS), lambda i: (0, 0)),
            pl.BlockSpec((N_TOK, EMBED_DIM), lambda i: (0, 0)),
        ],
        out_specs=pl.BlockSpec((N_TOK, EMBED_DIM), lambda i: (0, 0)),
        out_shape=jax.ShapeDtypeStruct((N_TOK, EMBED_DIM), jnp.float32),
    )(flat, w1, b1r, w2, b2r, idx3, wbT, ovf)

    return out.reshape(input.shape), aux[0, 0]


# speedup vs baseline: 1.1552x; 1.1552x over previous
import functools

import jax
import jax.numpy as jnp
from jax import lax
from jax.experimental import pallas as pl
from jax.experimental.pallas import tpu as pltpu
from jax.experimental.pallas import tpu_sc as plsc

EMBED_DIM = 768
FFN_DIM = 1536
NUM_EXPERTS = 8
CAPACITY = 512
ALPHA = 0.01
N_TOK = 2048
LANES = 128
TOK_TILE = 512
N_E = NUM_EXPERTS + 1
GRID = NUM_EXPERTS + N_TOK // TOK_TILE
OW_COL = 2 * NUM_EXPERTS


def _router_body(x_ref, rw_ref, rb_ref, wvT_ref, owT_ref, aux_ref):
    x = x_ref[...]
    logits = jnp.dot(x, rw_ref[...], preferred_element_type=jnp.float32)
    logits = logits + rb_ref[...]
    col = jax.lax.broadcasted_iota(jnp.int32, (N_TOK, NUM_EXPERTS), 1)
    m = jnp.max(logits, axis=1, keepdims=True)
    ex = jnp.exp(logits - m)
    s = ex / jnp.sum(ex, axis=1, keepdims=True)
    big_i = jnp.int32(LANES)
    m1 = jnp.max(s, axis=1, keepdims=True)
    i1 = jnp.min(jnp.where(s == m1, col, big_i), axis=1, keepdims=True)
    s2m = jnp.where(col == i1, -2.0, s)
    m2 = jnp.max(s2m, axis=1, keepdims=True)
    i2 = jnp.min(jnp.where(s2m == m2, col, big_i), axis=1, keepdims=True)
    wsum = m1 + m2
    wfull = (jnp.where(col == i1, m1 / wsum, 0.0)
             + jnp.where(col == i2, m2 / wsum, 0.0))
    maskf = jnp.where((col == i1) | (col == i2), 1.0, 0.0)
    rank = maskf
    k = 1
    while k < N_TOK:
        shifted = jnp.concatenate(
            [jnp.zeros((k, NUM_EXPERTS), jnp.float32), rank[: N_TOK - k, :]],
            axis=0)
        rank = rank + shifted
        k *= 2
    within = (maskf > 0.0) & (rank <= jnp.float32(CAPACITY))
    overflow = (maskf > 0.0) & ~within
    w_within = jnp.where(within, wfull, 0.0)
    ow_tok = jnp.sum(jnp.where(overflow, wfull, 0.0), axis=1, keepdims=True)
    nw = jnp.sum(within.astype(jnp.float32), axis=0, keepdims=True)
    p = jnp.sum(w_within, axis=0, keepdims=True) / jnp.maximum(nw, 1.0)
    f = nw / jnp.float32(N_TOK)
    aux_ref[...] = jnp.float32(ALPHA * NUM_EXPERTS) * jnp.sum(
        f * p, keepdims=True)
    destf = jnp.where(within, rank - 1.0, -1.0)
    wvT_ref[...] = jnp.concatenate([w_within, destf], axis=1).T
    owT_ref[...] = jnp.where(col == 0, ow_tok, 0.0)


def _sc_dispatch(flat_hbm, wvT_hbm,
                 xbuf_hbm, idx_hbm, wbuf_hbm,
                 wvec, dvec, idxv, wslotv, idx128, rows, sh_idx, sem):
    core = lax.axis_index("c")
    sub = lax.axis_index("s")
    lane = lax.iota(jnp.int32, 16)

    @pl.when(sub < 4)
    def _pack():
        e = core * 4 + sub
        pltpu.sync_copy(wvT_hbm.at[e], wvec)
        pltpu.sync_copy(wvT_hbm.at[NUM_EXPERTS + e], dvec)
        zi = jnp.zeros((16,), jnp.int32)
        zf = jnp.zeros((16,), jnp.float32)
        for i in range(CAPACITY // 16):
            idxv[pl.ds(i * 16, 16)] = zi
            wslotv[pl.ds(i * 16, 16)] = zf

        def body(i, carry):
            d16 = dvec[pl.ds(i * 16, 16)]
            within = d16 >= 0.0
            dest = jnp.maximum(d16, 0.0).astype(jnp.int32)
            w16 = wvec[pl.ds(i * 16, 16)]
            tok = lane + i * 16
            plsc.store_scatter(idxv, [dest], tok, mask=within)
            plsc.store_scatter(wslotv, [dest], w16, mask=within)
            return carry

        lax.fori_loop(0, N_TOK // 16, body, 0)
        pltpu.sync_copy(idxv, sh_idx.at[sub])
        pltpu.sync_copy(idxv, idx_hbm.at[e])
        pltpu.sync_copy(wslotv, wbuf_hbm.at[e])

    plsc.subcore_barrier()
    e_loc = sub // 4
    j = sub % 4
    e = core * 4 + e_loc
    pltpu.sync_copy(sh_idx.at[e_loc, pl.ds(j * 128, 128)], idx128)
    pltpu.async_copy(flat_hbm.at[idx128], rows, sem).wait()
    pltpu.sync_copy(rows, xbuf_hbm.at[pl.ds(e * CAPACITY + j * 128, 128)])


def _make_dispatch():
    return pl.kernel(
        _sc_dispatch,
        out_type=[
            jax.ShapeDtypeStruct((NUM_EXPERTS * CAPACITY, EMBED_DIM),
                                 jnp.float32),
            jax.ShapeDtypeStruct((NUM_EXPERTS, CAPACITY), jnp.int32),
            jax.ShapeDtypeStruct((NUM_EXPERTS, CAPACITY), jnp.float32),
        ],
        mesh=plsc.VectorSubcoreMesh(core_axis_name="c", subcore_axis_name="s"),
        compiler_params=pltpu.CompilerParams(needs_layout_passes=False),
        scratch_types=[
            pltpu.VMEM((N_TOK,), jnp.float32),
            pltpu.VMEM((N_TOK,), jnp.float32),
            pltpu.VMEM((CAPACITY,), jnp.int32),
            pltpu.VMEM((CAPACITY,), jnp.float32),
            pltpu.VMEM((128,), jnp.int32),
            pltpu.VMEM((128, EMBED_DIM), jnp.float32),
            pltpu.VMEM_SHARED((4, CAPACITY), jnp.int32),
            pltpu.SemaphoreType.DMA,
        ],
    )


def _ovf_body(xf_ref, ow1_ref, ob1_ref, ow2_ref, ob2_ref, owT_ref, out_ref):
    x = xf_ref[...].astype(jnp.bfloat16)
    h = jnp.dot(x, ow1_ref[...].astype(jnp.bfloat16),
                preferred_element_type=jnp.float32)
    h = jax.nn.relu(h + ob1_ref[...])
    y = jnp.dot(h.astype(jnp.bfloat16), ow2_ref[...].astype(jnp.bfloat16),
                preferred_element_type=jnp.float32)
    y = y + ob2_ref[...]
    c = jnp.sum(owT_ref[...], axis=1, keepdims=True)
    out_ref[...] = c * y


def _exp_body(xb_ref, w1_ref, b1_ref, w2_ref, b2_ref, idx_ref,
              wbT_ref, ovf_ref, out_ref):
    i = pl.program_id(0)

    @pl.when(i == 0)
    def _():
        out_ref[...] = ovf_ref[...]

    x = xb_ref[...].astype(jnp.bfloat16)
    h = jnp.dot(x, w1_ref[0].astype(jnp.bfloat16),
                preferred_element_type=jnp.float32)
    h = jax.nn.relu(h + b1_ref[0])
    y = jnp.dot(h.astype(jnp.bfloat16), w2_ref[0].astype(jnp.bfloat16),
                preferred_element_type=jnp.float32)
    y = y + b2_ref[0]
    lane8 = jax.lax.broadcasted_iota(jnp.int32, (CAPACITY, NUM_EXPERTS), 1)
    wcol = jnp.sum(jnp.where(lane8 == i, wbT_ref[...], 0.0),
                   axis=1, keepdims=True)
    ys = (wcol * y).astype(jnp.bfloat16)
    trow = jax.lax.broadcasted_iota(jnp.int32, (N_TOK, CAPACITY), 0)
    oh = (trow == idx_ref[0]).astype(jnp.bfloat16)
    out_ref[...] += jnp.dot(oh, ys, preferred_element_type=jnp.float32)


def kernel(input, router_w, router_b, w1, b1, w2, b2, ow1, ob1, ow2, ob2):
    flat = input.reshape(N_TOK, EMBED_DIM)
    rb_p = router_b.reshape(1, NUM_EXPERTS)

    wvT, owT, aux = pl.pallas_call(
        _router_body,
        out_shape=[
            jax.ShapeDtypeStruct((2 * NUM_EXPERTS, N_TOK), jnp.float32),
            jax.ShapeDtypeStruct((N_TOK, NUM_EXPERTS), jnp.float32),
            jax.ShapeDtypeStruct((1, 1), jnp.float32),
        ],
    )(flat, router_w, rb_p)

    xbuf, idx, wbuf = _make_dispatch()(flat, wvT)

    idx3 = idx.reshape(NUM_EXPERTS, 1, CAPACITY)
    wbT = wbuf.T

    b1r = b1.reshape(NUM_EXPERTS, 1, FFN_DIM)
    b2r = b2.reshape(NUM_EXPERTS, 1, EMBED_DIM)
    ob1r = ob1.reshape(1, FFN_DIM)
    ob2r = ob2.reshape(1, EMBED_DIM)

    ovf = pl.pallas_call(
        _ovf_body,
        grid=(N_TOK // TOK_TILE,),
        in_specs=[
            pl.BlockSpec((TOK_TILE, EMBED_DIM), lambda t: (t, 0)),
            pl.BlockSpec((EMBED_DIM, FFN_DIM), lambda t: (0, 0)),
            pl.BlockSpec((1, FFN_DIM), lambda t: (0, 0)),
            pl.BlockSpec((FFN_DIM, EMBED_DIM), lambda t: (0, 0)),
            pl.BlockSpec((1, EMBED_DIM), lambda t: (0, 0)),
            pl.BlockSpec((TOK_TILE, NUM_EXPERTS), lambda t: (t, 0)),
        ],
        out_specs=pl.BlockSpec((TOK_TILE, EMBED_DIM), lambda t: (t, 0)),
        out_shape=jax.ShapeDtypeStruct((N_TOK, EMBED_DIM), jnp.float32),
    )(flat, ow1, ob1r, ow2, ob2r, owT)

    out = pl.pallas_call(
        _exp_body,
        grid=(NUM_EXPERTS,),
        in_specs=[
            pl.BlockSpec((TOK_TILE, EMBED_DIM), lambda i: (i, 0)),
            pl.BlockSpec((1, EMBED_DIM, FFN_DIM), lambda i: (i, 0, 0)),
            pl.BlockSpec((1, 1, FFN_DIM), lambda i: (i, 0, 0)),
            pl.BlockSpec((1, FFN_DIM, EMBED_DIM), lambda i: (i, 0, 0)),
            pl.BlockSpec((1, 1, EMBED_DIM), lambda i: (i, 0, 0)),
            pl.BlockSpec((1, 1, CAPACITY), lambda i: (i, 0, 0)),
            pl.BlockSpec((CAPACITY, NUM_EXPERTS), lambda i: (0, 0)),
            pl.BlockSpec((N_TOK, EMBED_DIM), lambda i: (0, 0)),
        ],
        out_specs=pl.BlockSpec((N_TOK, EMBED_DIM), lambda i: (0, 0)),
        out_shape=jax.ShapeDtypeStruct((N_TOK, EMBED_DIM), jnp.float32),
    )(xbuf, w1, b1r, w2, b2r, idx3, wbT, ovf)

    return out.reshape(input.shape), aux[0, 0]
